# Initial kernel scaffold; baseline (speedup 1.0000x reference)
import jax
import jax.numpy as jnp
from jax.experimental import pallas as pl


def kernel(inputs, edge_index, edge_type, node_feature, W1, Ws1, b1, W2, Ws2, b2, Wf1, bf1, Wf2, bf2, Wf3, bf3):
    raise NotImplementedError("write your pallas kernel here")


# speedup vs baseline: 2.9195x; 2.9195x over previous
import functools

import jax
import jax.numpy as jnp
from jax import lax
from jax.experimental import pallas as pl
from jax.experimental.pallas import tpu as pltpu
from jax.experimental.pallas import tpu_sc as plsc

NC = 2
NS = 16
NW = NC * NS


def _make_agg(V, EP, NRP, with_cnt, interpret=False):
  JT = EP // 128 // NS
  RT = NRP // NS
  mesh = plsc.VectorSubcoreMesh(
      core_axis_name="c", subcore_axis_name="s",
      num_cores=NC, num_subcores=NS)

  out_type = [jax.ShapeDtypeStruct((4, NRP, 32), jnp.float32)]
  scratch = [
      pltpu.VMEM((JT, 128), jnp.int32),
      pltpu.VMEM((JT, 128), jnp.int32),
      pltpu.VMEM((128, 32), jnp.float32),
      pltpu.VMEM_SHARED((NRP, 32), jnp.float32),
      pltpu.SemaphoreType.DMA,
  ]
  if with_cnt:
    out_type.append(jax.ShapeDtypeStruct((2, NRP, 32), jnp.float32))
    scratch.append(pltpu.VMEM((128, 32), jnp.float32))

  def body(t0, t1, t2, t3, gidx, sidx, z32, *rest):
    if with_cnt:
      s4, cnt2, gv, sv, rows, acc, sem, ones_v = rest
    else:
      s4, gv, sv, rows, acc, sem = rest
    tables = (t0, t1, t2, t3)
    c = lax.axis_index("c")
    s = lax.axis_index("s")
    row0 = s * RT

    pltpu.sync_copy(gidx.at[pl.ds(s * JT, JT)], gv)
    pltpu.sync_copy(sidx.at[pl.ds(s * JT, JT)], sv)
    if with_cnt:
      def fill(i, carry):
        ones_v[i, pl.ds(0, 16)] = jnp.full((16,), 1.0, jnp.float32)
        ones_v[i, pl.ds(16, 16)] = jnp.full((16,), 1.0, jnp.float32)
        return carry
      lax.fori_loop(0, 128, fill, 0)

    def zero_acc():
      pltpu.sync_copy(z32, acc.at[pl.ds(row0, RT)])
      plsc.subcore_barrier()

    def do_pass(tbl, cc):
      zero_acc()
      def step(j, carry):
        pltpu.async_copy(tbl.at[gv.at[j]], rows, sem).wait()
        pltpu.sync_copy(rows, acc.at[sv.at[j]], add=True)
        return carry
      lax.fori_loop(0, JT, step, 0)
      plsc.subcore_barrier()
      pltpu.sync_copy(acc.at[pl.ds(row0, RT)], s4.at[cc, pl.ds(row0, RT)])
      plsc.subcore_barrier()

    def do_cnt_pass(k):
      zero_acc()
      def step(j, carry):
        pltpu.sync_copy(ones_v, acc.at[sv.at[k * (JT // 2) + j]], add=True)
        return carry
      lax.fori_loop(0, JT // 2, step, 0)
      plsc.subcore_barrier()
      pltpu.sync_copy(acc.at[pl.ds(row0, RT)], cnt2.at[k, pl.ds(row0, RT)])
      plsc.subcore_barrier()

    for p in range(2):
      for k in range(NC):
        @pl.when(c == k)
        def _(p=p, k=k):
          do_pass(tables[2 * p + k], 2 * p + k)
    if with_cnt:
      for k in range(NC):
        @pl.when(c == k)
        def _(k=k):
          do_cnt_pass(k)

  return pl.kernel(body, out_type=out_type if with_cnt else out_type[0],
                   mesh=mesh, scratch_types=scratch, interpret=interpret,
                   compiler_params=pltpu.CompilerParams(
                       use_tc_tiling_on_sc=False))


def _make_gather(BIDX, D, interpret=False):
  rows_per_w = BIDX // NW
  JW = rows_per_w // 128
  mesh = plsc.VectorSubcoreMesh(
      core_axis_name="c", subcore_axis_name="s",
      num_cores=NC, num_subcores=NS)
  scratch = [
      pltpu.VMEM((BIDX // 128, 128), jnp.int32),
      pltpu.VMEM((128, D), jnp.float32),
      pltpu.SemaphoreType.DMA,
  ]

  def body(table, idx2, out, iv, rv, sem):
    c = lax.axis_index("c")
    s = lax.axis_index("s")
    w = s * NC + c
    pltpu.sync_copy(idx2, iv)
    for j in range(JW):
      pltpu.async_copy(table.at[iv.at[w * JW + j]], rv, sem).wait()
      pltpu.sync_copy(rv, out.at[pl.ds(w * rows_per_w + j * 128, 128)])

  return pl.kernel(
      body, out_type=jax.ShapeDtypeStruct((BIDX, D), jnp.float32),
      mesh=mesh, scratch_types=scratch, interpret=interpret,
      compiler_params=pltpu.CompilerParams(use_tc_tiling_on_sc=False))


def _tc_a(S1, cnta, cntb, x, W1r, Ws1, b1, W2r, Ws2, interpret=False):
  N, R, D = S1.shape
  H = Ws1.shape[1]
  LAST = Ws2.shape[1]
  bn = 1000
  grid = N // bn

  def body(s1_ref, cnta_ref, cntb_ref, x_ref, w1_ref, ws1_ref, b1_ref,
           w2_ref, ws2_ref, g2_ref, self2_ref):
    cnt = cnta_ref[:, :, 0:1] + cntb_ref[:, :, 0:1]
    inv = 1.0 / jnp.maximum(cnt, 1.0)
    acc = jnp.dot(x_ref[...], ws1_ref[...],
                  preferred_element_type=jnp.float32) + b1_ref[...]
    for r in range(R):
      mean_r = s1_ref[:, r, :] * inv[:, r]
      acc = acc + jnp.dot(mean_r, w1_ref[r],
                          preferred_element_type=jnp.float32)
    h = jnp.maximum(acc, 0.0)
    self2_ref[...] = jnp.dot(h, ws2_ref[...],
                             preferred_element_type=jnp.float32)
    for r in range(R):
      g2_ref[:, r, :] = jnp.dot(h, w2_ref[r],
                                preferred_element_type=jnp.float32)

  return pl.pallas_call(
      body,
      grid=(grid,),
      in_specs=[
          pl.BlockSpec((bn, R, D), lambda i: (i, 0, 0)),
          pl.BlockSpec((bn, R, 32), lambda i: (i, 0, 0)),
          pl.BlockSpec((bn, R, 32), lambda i: (i, 0, 0)),
          pl.BlockSpec((bn, D), lambda i: (i, 0)),
          pl.BlockSpec((R, D, H), lambda i: (0, 0, 0)),
          pl.BlockSpec((D, H), lambda i: (0, 0)),
          pl.BlockSpec((1, H), lambda i: (0, 0)),
          pl.BlockSpec((R, H, LAST), lambda i: (0, 0, 0)),
          pl.BlockSpec((H, LAST), lambda i: (0, 0)),
      ],
      out_specs=[
          pl.BlockSpec((bn, R, LAST), lambda i: (i, 0, 0)),
          pl.BlockSpec((bn, LAST), lambda i: (i, 0)),
      ],
      out_shape=[
          jax.ShapeDtypeStruct((N, R, LAST), jnp.float32),
          jax.ShapeDtypeStruct((N, LAST), jnp.float32),
      ],
      interpret=interpret,
  )(S1, cnta, cntb, x, W1r, Ws1, b1, W2r, Ws2)


def _tc_b(S2, cnta, cntb, self2, b2, interpret=False):
  N, R, LAST = S2.shape
  bn = 1000
  grid = N // bn

  def body(s2_ref, cnta_ref, cntb_ref, self2_ref, b2_ref, out_ref):
    cnt = cnta_ref[:, :, 0:1] + cntb_ref[:, :, 0:1]
    inv = 1.0 / jnp.maximum(cnt, 1.0)
    acc = self2_ref[...] + b2_ref[...]
    for r in range(R):
      acc = acc + s2_ref[:, r, :] * inv[:, r]
    out_ref[...] = jnp.maximum(acc, 0.0)

  return pl.pallas_call(
      body,
      grid=(grid,),
      in_specs=[
          pl.BlockSpec((bn, R, LAST), lambda i: (i, 0, 0)),
          pl.BlockSpec((bn, R, 32), lambda i: (i, 0, 0)),
          pl.BlockSpec((bn, R, 32), lambda i: (i, 0, 0)),
          pl.BlockSpec((bn, LAST), lambda i: (i, 0)),
          pl.BlockSpec((1, LAST), lambda i: (0, 0)),
      ],
      out_specs=pl.BlockSpec((bn, LAST), lambda i: (i, 0)),
      out_shape=jax.ShapeDtypeStruct((N, LAST), jnp.float32),
      interpret=interpret,
  )(S2, cnta, cntb, self2, b2)


def _tc_c(x1, x2, Wf1a, Wf1b, bf1, Wf2, bf2, Wf3p, bf3p, interpret=False):
  B, LAST = x1.shape
  H1 = Wf1a.shape[1]
  H2 = Wf2.shape[1]
  OP = Wf3p.shape[1]
  bn = 1024
  grid = B // bn

  def body(x1_ref, x2_ref, a_ref, b_ref, b1_ref, w2_ref, b2_ref, w3_ref,
           b3_ref, out_ref):
    y = jnp.dot(x1_ref[...], a_ref[...], preferred_element_type=jnp.float32)
    y = y + jnp.dot(x2_ref[...], b_ref[...],
                    preferred_element_type=jnp.float32)
    y = jnp.maximum(y + b1_ref[...], 0.0)
    y = jnp.maximum(jnp.dot(y, w2_ref[...],
                            preferred_element_type=jnp.float32) + b2_ref[...],
                    0.0)
    out_ref[...] = jnp.dot(y, w3_ref[...],
                           preferred_element_type=jnp.float32) + b3_ref[...]

  return pl.pallas_call(
      body,
      grid=(grid,),
      in_specs=[
          pl.BlockSpec((bn, LAST), lambda i: (i, 0)),
          pl.BlockSpec((bn, LAST), lambda i: (i, 0)),
          pl.BlockSpec((LAST, H1), lambda i: (0, 0)),
          pl.BlockSpec((LAST, H1), lambda i: (0, 0)),
          pl.BlockSpec((1, H1), lambda i: (0, 0)),
          pl.BlockSpec((H1, H2), lambda i: (0, 0)),
          pl.BlockSpec((1, H2), lambda i: (0, 0)),
          pl.BlockSpec((H2, OP), lambda i: (0, 0)),
          pl.BlockSpec((1, OP), lambda i: (0, 0)),
      ],
      out_specs=pl.BlockSpec((bn, OP), lambda i: (i, 0)),
      out_shape=jax.ShapeDtypeStruct((B, OP), jnp.float32),
      interpret=interpret,
  )(x1, x2, Wf1a, Wf1b, bf1, Wf2, bf2, Wf3p, bf3p)


def kernel(inputs, edge_index, edge_type, node_feature,
           W1, Ws1, b1, W2, Ws2, b2,
           Wf1, bf1, Wf2, bf2, Wf3, bf3):
  N, D = node_feature.shape
  H = Ws1.shape[1]
  R = W1.shape[0] // D
  LAST = W2.shape[1]
  E = edge_type.shape[0]
  B = inputs.shape[0]
  NR = N * R

  NRP = -(-NR // (NS * 128)) * (NS * 128)
  EP = -(-E // (NS * 128 * 8)) * (NS * 128 * 8)
  npad = EP - E

  src = edge_index[0].astype(jnp.int32)
  dst = edge_index[1].astype(jnp.int32)
  rel = edge_type.astype(jnp.int32)

  pad_g = jnp.arange(npad, dtype=jnp.int32) % N
  pad_s = NR + jnp.arange(npad, dtype=jnp.int32) % (NRP - NR)
  sidx = jnp.concatenate([dst * R + rel, pad_s]).reshape(EP // 128, 128)
  g1 = jnp.concatenate([src, pad_g]).reshape(EP // 128, 128)
  g2 = jnp.concatenate([src * R + rel, pad_g]).reshape(EP // 128, 128)

  z32 = jnp.zeros((NRP // NS, 32), jnp.float32)

  x4 = node_feature.reshape(N, 4, 32).transpose(1, 0, 2)
  agg1 = _make_agg(N, EP, NRP, with_cnt=True)
  S4_1, cnt2 = agg1(x4[0], x4[1], x4[2], x4[3], g1, sidx, z32)
  S1 = S4_1.transpose(1, 0, 2).reshape(NRP, D)[:NR].reshape(N, R, D)
  cnta = cnt2[0, :NR].reshape(N, R, 32)
  cntb = cnt2[1, :NR].reshape(N, R, 32)

  G2, self2 = _tc_a(S1, cnta, cntb, node_feature,
                    W1.reshape(R, D, H), Ws1, b1.reshape(1, H),
                    W2.reshape(R, H, LAST), Ws2)

  G2f = G2.reshape(NR, LAST)
  agg2 = _make_agg(NR, EP, NRP, with_cnt=False)
  S4_2 = agg2(G2f[:, 0:32], G2f[:, 32:64], G2f[:, 64:96], G2f[:, 96:128],
              g2, sidx, z32)
  S2 = S4_2.transpose(1, 0, 2).reshape(NRP, LAST)[:NR].reshape(N, R, LAST)

  h2 = _tc_b(S2, cnta, cntb, self2, b2.reshape(1, LAST))

  idx_all = jnp.concatenate(
      [inputs[:, 0], inputs[:, 1]]).astype(jnp.int32).reshape(-1, 128)
  rows = _make_gather(2 * B, LAST)(h2, idx_all)
  x1, x2 = rows[:B], rows[B:]

  OP = 8
  Wf3p = jnp.concatenate(
      [Wf3, jnp.zeros((Wf3.shape[0], OP - 1), jnp.float32)], axis=1)
  bf3p = jnp.concatenate(
      [bf3, jnp.zeros((OP - 1,), jnp.float32)]).reshape(1, OP)
  out = _tc_c(x1, x2, Wf1[:LAST], Wf1[LAST:], bf1.reshape(1, -1),
              Wf2, bf2.reshape(1, -1), Wf3p, bf3p)
  return out[:, 0:1]


# speedup vs baseline: 4.1758x; 1.4303x over previous
import functools

import jax
import jax.numpy as jnp
from jax import lax
from jax.experimental import pallas as pl
from jax.experimental.pallas import tpu as pltpu
from jax.experimental.pallas import tpu_sc as plsc

NC = 2
NS = 16
NW = NC * NS


def _make_agg(V, EP, NRP, with_cnt, interpret=False):
  JT = EP // 128 // NS
  RT = NRP // NS
  mesh = plsc.VectorSubcoreMesh(
      core_axis_name="c", subcore_axis_name="s",
      num_cores=NC, num_subcores=NS)

  out_type = [jax.ShapeDtypeStruct((NRP, 128), jnp.float32)]
  scratch = [
      pltpu.VMEM((JT, 128), jnp.int32),
      pltpu.VMEM((JT, 128), jnp.int32),
      pltpu.VMEM((128, 32), jnp.float32),
      pltpu.VMEM((128, 32), jnp.float32),
      pltpu.VMEM_SHARED((NRP, 32), jnp.float32),
      pltpu.SemaphoreType.DMA,
  ]
  if with_cnt:
    out_type.append(jax.ShapeDtypeStruct((2, NRP, 32), jnp.float32))
    scratch.append(pltpu.VMEM((128, 32), jnp.float32))

  def body(t0, t1, t2, t3, gidx, sidx, z32, *rest):
    if with_cnt:
      s4, cnt2, gv, sv, rows0, rows1, acc, sem, ones_v = rest
    else:
      s4, gv, sv, rows0, rows1, acc, sem = rest
    tables = (t0, t1, t2, t3)
    c = lax.axis_index("c")
    s = lax.axis_index("s")
    row0 = s * RT

    pltpu.sync_copy(gidx.at[pl.ds(s * JT, JT)], gv)
    pltpu.sync_copy(sidx.at[pl.ds(s * JT, JT)], sv)
    if with_cnt:
      def fill(i, carry):
        ones_v[i, pl.ds(0, 16)] = jnp.full((16,), 1.0, jnp.float32)
        ones_v[i, pl.ds(16, 16)] = jnp.full((16,), 1.0, jnp.float32)
        return carry
      lax.fori_loop(0, 128, fill, 0)

    def zero_acc():
      pltpu.sync_copy(z32, acc.at[pl.ds(row0, RT)])
      plsc.subcore_barrier()

    def do_pass(tbl, cc):
      zero_acc()
      pltpu.async_copy(tbl.at[gv.at[0]], rows0, sem)
      def step(jj, carry):
        j0 = 2 * jj
        j1 = j0 + 1
        jn = jnp.minimum(j0 + 2, JT - 1)
        pltpu.make_async_copy(tbl.at[gv.at[j0]], rows0, sem).wait()
        pltpu.async_copy(tbl.at[gv.at[j1]], rows1, sem)
        pltpu.sync_copy(rows0, acc.at[sv.at[j0]], add=True)
        pltpu.make_async_copy(tbl.at[gv.at[j1]], rows1, sem).wait()
        pltpu.async_copy(tbl.at[gv.at[jn]], rows0, sem)
        pltpu.sync_copy(rows1, acc.at[sv.at[j1]], add=True)
        return carry
      lax.fori_loop(0, JT // 2, step, 0)
      pltpu.make_async_copy(tbl.at[gv.at[JT - 1]], rows0, sem).wait()
      plsc.subcore_barrier()
      pltpu.sync_copy(acc.at[pl.ds(row0, RT)],
                      s4.at[pl.ds(row0, RT), pl.ds(cc * 32, 32)])
      plsc.subcore_barrier()

    def do_cnt_pass(k):
      zero_acc()
      def step(j, carry):
        pltpu.sync_copy(ones_v, acc.at[sv.at[k * (JT // 2) + j]], add=True)
        return carry
      lax.fori_loop(0, JT // 2, step, 0)
      plsc.subcore_barrier()
      pltpu.sync_copy(acc.at[pl.ds(row0, RT)], cnt2.at[k, pl.ds(row0, RT)])
      plsc.subcore_barrier()

    for p in range(2):
      for k in range(NC):
        @pl.when(c == k)
        def _(p=p, k=k):
          do_pass(tables[2 * p + k], 2 * p + k)
    if with_cnt:
      for k in range(NC):
        @pl.when(c == k)
        def _(k=k):
          do_cnt_pass(k)

  return pl.kernel(body, out_type=out_type if with_cnt else out_type[0],
                   mesh=mesh, scratch_types=scratch, interpret=interpret,
                   compiler_params=pltpu.CompilerParams(
                       use_tc_tiling_on_sc=False))


def _make_gather(BIDX, D, interpret=False):
  rows_per_w = BIDX // NW
  JW = rows_per_w // 128
  mesh = plsc.VectorSubcoreMesh(
      core_axis_name="c", subcore_axis_name="s",
      num_cores=NC, num_subcores=NS)
  scratch = [
      pltpu.VMEM((BIDX // 128, 128), jnp.int32),
      pltpu.VMEM((128, D), jnp.float32),
      pltpu.SemaphoreType.DMA,
  ]

  def body(table, idx2, out, iv, rv, sem):
    c = lax.axis_index("c")
    s = lax.axis_index("s")
    w = s * NC + c
    pltpu.sync_copy(idx2, iv)
    for j in range(JW):
      pltpu.async_copy(table.at[iv.at[w * JW + j]], rv, sem).wait()
      pltpu.sync_copy(rv, out.at[pl.ds(w * rows_per_w + j * 128, 128)])

  return pl.kernel(
      body, out_type=jax.ShapeDtypeStruct((BIDX, D), jnp.float32),
      mesh=mesh, scratch_types=scratch, interpret=interpret,
      compiler_params=pltpu.CompilerParams(use_tc_tiling_on_sc=False))


def _tc_a(S1, cnta, cntb, x, W1r, Ws1, b1, W2r, Ws2, interpret=False):
  NP, R, D = S1.shape
  H = Ws1.shape[1]
  LAST = Ws2.shape[1]
  bn = 1024
  grid = NP // bn

  def body(s1_ref, cnta_ref, cntb_ref, x_ref, w1_ref, ws1_ref, b1_ref,
           w2_ref, ws2_ref, g2c0_ref, g2c1_ref, g2c2_ref, g2c3_ref,
           self2_ref):
    cnt = cnta_ref[:, :, 0:1] + cntb_ref[:, :, 0:1]
    inv = 1.0 / jnp.maximum(cnt, 1.0)
    acc = jnp.dot(x_ref[...], ws1_ref[...],
                  preferred_element_type=jnp.float32) + b1_ref[...]
    for r in range(R):
      mean_r = s1_ref[:, r, :] * inv[:, r]
      acc = acc + jnp.dot(mean_r, w1_ref[r],
                          preferred_element_type=jnp.float32)
    h = jnp.maximum(acc, 0.0)
    self2_ref[...] = jnp.dot(h, ws2_ref[...],
                             preferred_element_type=jnp.float32)
    chunks = (g2c0_ref, g2c1_ref, g2c2_ref, g2c3_ref)
    for r in range(R):
      hw = jnp.dot(h, w2_ref[r], preferred_element_type=jnp.float32)
      for c4 in range(4):
        chunks[c4][:, r, :] = hw[:, c4 * 32:(c4 + 1) * 32]

  cspec = pl.BlockSpec((bn, R, 32), lambda i: (i, 0, 0))
  return pl.pallas_call(
      body,
      grid=(grid,),
      in_specs=[
          pl.BlockSpec((bn, R, D), lambda i: (i, 0, 0)),
          cspec,
          cspec,
          pl.BlockSpec((bn, D), lambda i: (i, 0)),
          pl.BlockSpec((R, D, H), lambda i: (0, 0, 0)),
          pl.BlockSpec((D, H), lambda i: (0, 0)),
          pl.BlockSpec((1, H), lambda i: (0, 0)),
          pl.BlockSpec((R, H, LAST), lambda i: (0, 0, 0)),
          pl.BlockSpec((H, LAST), lambda i: (0, 0)),
      ],
      out_specs=[cspec, cspec, cspec, cspec,
                 pl.BlockSpec((bn, LAST), lambda i: (i, 0))],
      out_shape=[jax.ShapeDtypeStruct((NP, R, 32), jnp.float32)] * 4 +
                [jax.ShapeDtypeStruct((NP, LAST), jnp.float32)],
      interpret=interpret,
  )(S1, cnta, cntb, x, W1r, Ws1, b1, W2r, Ws2)


def _tc_b(S2, cnta, cntb, self2, b2, interpret=False):
  NP, R, LAST = S2.shape
  bn = 1024
  grid = NP // bn

  def body(s2_ref, cnta_ref, cntb_ref, self2_ref, b2_ref, out_ref):
    cnt = cnta_ref[:, :, 0:1] + cntb_ref[:, :, 0:1]
    inv = 1.0 / jnp.maximum(cnt, 1.0)
    acc = self2_ref[...] + b2_ref[...]
    for r in range(R):
      acc = acc + s2_ref[:, r, :] * inv[:, r]
    out_ref[...] = jnp.maximum(acc, 0.0)

  return pl.pallas_call(
      body,
      grid=(grid,),
      in_specs=[
          pl.BlockSpec((bn, R, LAST), lambda i: (i, 0, 0)),
          pl.BlockSpec((bn, R, 32), lambda i: (i, 0, 0)),
          pl.BlockSpec((bn, R, 32), lambda i: (i, 0, 0)),
          pl.BlockSpec((bn, LAST), lambda i: (i, 0)),
          pl.BlockSpec((1, LAST), lambda i: (0, 0)),
      ],
      out_specs=pl.BlockSpec((bn, LAST), lambda i: (i, 0)),
      out_shape=jax.ShapeDtypeStruct((NP, LAST), jnp.float32),
      interpret=interpret,
  )(S2, cnta, cntb, self2, b2)


def _tc_c(x1, x2, Wf1a, Wf1b, bf1, Wf2, bf2, Wf3p, bf3p, interpret=False):
  B, LAST = x1.shape
  H1 = Wf1a.shape[1]
  H2 = Wf2.shape[1]
  OP = Wf3p.shape[1]
  bn = 1024
  grid = B // bn

  def body(x1_ref, x2_ref, a_ref, b_ref, b1_ref, w2_ref, b2_ref, w3_ref,
           b3_ref, out_ref):
    y = jnp.dot(x1_ref[...], a_ref[...], preferred_element_type=jnp.float32)
    y = y + jnp.dot(x2_ref[...], b_ref[...],
                    preferred_element_type=jnp.float32)
    y = jnp.maximum(y + b1_ref[...], 0.0)
    y = jnp.maximum(jnp.dot(y, w2_ref[...],
                            preferred_element_type=jnp.float32) + b2_ref[...],
                    0.0)
    out_ref[...] = jnp.dot(y, w3_ref[...],
                           preferred_element_type=jnp.float32) + b3_ref[...]

  return pl.pallas_call(
      body,
      grid=(grid,),
      in_specs=[
          pl.BlockSpec((bn, LAST), lambda i: (i, 0)),
          pl.BlockSpec((bn, LAST), lambda i: (i, 0)),
          pl.BlockSpec((LAST, H1), lambda i: (0, 0)),
          pl.BlockSpec((LAST, H1), lambda i: (0, 0)),
          pl.BlockSpec((1, H1), lambda i: (0, 0)),
          pl.BlockSpec((H1, H2), lambda i: (0, 0)),
          pl.BlockSpec((1, H2), lambda i: (0, 0)),
          pl.BlockSpec((H2, OP), lambda i: (0, 0)),
          pl.BlockSpec((1, OP), lambda i: (0, 0)),
      ],
      out_specs=pl.BlockSpec((bn, OP), lambda i: (i, 0)),
      out_shape=jax.ShapeDtypeStruct((B, OP), jnp.float32),
      interpret=interpret,
  )(x1, x2, Wf1a, Wf1b, bf1, Wf2, bf2, Wf3p, bf3p)


def kernel(inputs, edge_index, edge_type, node_feature,
           W1, Ws1, b1, W2, Ws2, b2,
           Wf1, bf1, Wf2, bf2, Wf3, bf3):
  N, D = node_feature.shape
  H = Ws1.shape[1]
  R = W1.shape[0] // D
  LAST = W2.shape[1]
  E = edge_type.shape[0]
  B = inputs.shape[0]
  NR = N * R

  NRP = -(-NR // (NS * 128)) * (NS * 128)
  EP = -(-E // (NS * 128 * 8)) * (NS * 128 * 8)
  npad = EP - E

  src = edge_index[0].astype(jnp.int32)
  dst = edge_index[1].astype(jnp.int32)
  rel = edge_type.astype(jnp.int32)

  pad_g = jnp.arange(npad, dtype=jnp.int32) % N
  pad_s = NR + jnp.arange(npad, dtype=jnp.int32) % (NRP - NR)
  sidx = jnp.concatenate([dst * R + rel, pad_s]).reshape(EP // 128, 128)
  g1 = jnp.concatenate([src, pad_g]).reshape(EP // 128, 128)
  g2 = jnp.concatenate([src * R + rel, pad_g]).reshape(EP // 128, 128)

  z32 = jnp.zeros((NRP // NS, 32), jnp.float32)
  NP = NRP // R

  x4 = node_feature.reshape(N, 4, 32).transpose(1, 0, 2)
  agg1 = _make_agg(N, EP, NRP, with_cnt=True)
  S4_1, cnt2 = agg1(x4[0], x4[1], x4[2], x4[3], g1, sidx, z32)
  S1 = S4_1.reshape(NP, R, D)
  cnta = cnt2[0].reshape(NP, R, 32)
  cntb = cnt2[1].reshape(NP, R, 32)
  xp = jnp.concatenate(
      [node_feature, jnp.zeros((NP - N, D), jnp.float32)])

  g2c0, g2c1, g2c2, g2c3, self2 = _tc_a(
      S1, cnta, cntb, xp,
      W1.reshape(R, D, H), Ws1, b1.reshape(1, H),
      W2.reshape(R, H, LAST), Ws2)

  agg2 = _make_agg(NRP, EP, NRP, with_cnt=False)
  S4_2 = agg2(g2c0.reshape(NRP, 32), g2c1.reshape(NRP, 32),
              g2c2.reshape(NRP, 32), g2c3.reshape(NRP, 32),
              g2, sidx, z32)
  S2 = S4_2.reshape(NP, R, LAST)

  h2 = _tc_b(S2, cnta, cntb, self2, b2.reshape(1, LAST))

  idx_all = jnp.concatenate(
      [inputs[:, 0], inputs[:, 1]]).astype(jnp.int32).reshape(-1, 128)
  rows = _make_gather(2 * B, LAST)(h2, idx_all)
  x1, x2 = rows[:B], rows[B:]

  OP = 8
  Wf3p = jnp.concatenate(
      [Wf3, jnp.zeros((Wf3.shape[0], OP - 1), jnp.float32)], axis=1)
  bf3p = jnp.concatenate(
      [bf3, jnp.zeros((OP - 1,), jnp.float32)]).reshape(1, OP)
  out = _tc_c(x1, x2, Wf1[:LAST], Wf1[LAST:], bf1.reshape(1, -1),
              Wf2, bf2.reshape(1, -1), Wf3p, bf3p)
  return out[:, 0:1]


# speedup vs baseline: 6.3716x; 1.5258x over previous
import functools

import jax
import jax.numpy as jnp
from jax import lax
from jax.experimental import pallas as pl
from jax.experimental.pallas import tpu as pltpu
from jax.experimental.pallas import tpu_sc as plsc

NC = 2
NS = 16
NW = NC * NS


def _make_agg(V, EP, NRP, with_cnt, interpret=False):
  JT = EP // 128 // NS
  RT = NRP // NS
  mesh = plsc.VectorSubcoreMesh(
      core_axis_name="c", subcore_axis_name="s",
      num_cores=NC, num_subcores=NS)

  out_type = [jax.ShapeDtypeStruct((NRP, 128), jnp.float32)]
  scratch = [
      pltpu.VMEM((JT, 128), jnp.int32),
      pltpu.VMEM((JT, 128), jnp.int32),
      pltpu.VMEM((JT, 128), jnp.int32),
      pltpu.VMEM((128, 32), jnp.float32),
      pltpu.VMEM((128, 32), jnp.float32),
      pltpu.VMEM_SHARED((NRP, 32), jnp.float32),
      pltpu.SemaphoreType.DMA,
  ]
  if with_cnt:
    out_type.append(jax.ShapeDtypeStruct((NRP, 128), jnp.float32))
    scratch.append(pltpu.VMEM((128, 32), jnp.float32))

  def body(tbl, gidx, sidx, z32, *rest):
    if with_cnt:
      s4, cnt, gv, sv, gvc, rows0, rows1, acc, sem, ones_v = rest
    else:
      s4, gv, sv, gvc, rows0, rows1, acc, sem = rest
    c = lax.axis_index("c")
    s = lax.axis_index("s")
    row0 = s * RT

    pltpu.sync_copy(gidx.at[pl.ds(s * JT, JT)], gv)
    pltpu.sync_copy(sidx.at[pl.ds(s * JT, JT)], sv)
    if with_cnt:
      def fill(i, carry):
        ones_v[i, pl.ds(0, 16)] = jnp.full((16,), 1.0, jnp.float32)
        ones_v[i, pl.ds(16, 16)] = jnp.full((16,), 1.0, jnp.float32)
        return carry
      lax.fori_loop(0, 128, fill, 0)

    def zero_acc():
      pltpu.sync_copy(z32, acc.at[pl.ds(row0, RT)])
      plsc.subcore_barrier()

    def do_pass(cc):
      def adjust(j, carry):
        for l in range(8):
          sl = pl.ds(l * 16, 16)
          gvc[j, sl] = gv[j, sl] * 4 + cc
        return carry
      lax.fori_loop(0, JT, adjust, 0)
      zero_acc()
      pltpu.async_copy(tbl.at[gvc.at[0]], rows0, sem)
      def step(jj, carry):
        j0 = 2 * jj
        j1 = j0 + 1
        jn = jnp.minimum(j0 + 2, JT - 1)
        pltpu.make_async_copy(tbl.at[gvc.at[j0]], rows0, sem).wait()
        pltpu.async_copy(tbl.at[gvc.at[j1]], rows1, sem)
        pltpu.sync_copy(rows0, acc.at[sv.at[j0]], add=True)
        pltpu.make_async_copy(tbl.at[gvc.at[j1]], rows1, sem).wait()
        pltpu.async_copy(tbl.at[gvc.at[jn]], rows0, sem)
        pltpu.sync_copy(rows1, acc.at[sv.at[j1]], add=True)
        return carry
      lax.fori_loop(0, JT // 2, step, 0)
      pltpu.make_async_copy(tbl.at[gvc.at[JT - 1]], rows0, sem).wait()
      plsc.subcore_barrier()
      pltpu.sync_copy(acc.at[pl.ds(row0, RT)],
                      s4.at[pl.ds(row0, RT), pl.ds(cc * 32, 32)])
      plsc.subcore_barrier()

    def do_cnt_pass(k):
      zero_acc()
      def step(j, carry):
        pltpu.sync_copy(ones_v, acc.at[sv.at[k * (JT // 2) + j]], add=True)
        return carry
      lax.fori_loop(0, JT // 2, step, 0)
      plsc.subcore_barrier()
      pltpu.sync_copy(acc.at[pl.ds(row0, RT)],
                      cnt.at[pl.ds(row0, RT), pl.ds(k * 32, 32)])
      plsc.subcore_barrier()

    for p in range(2):
      for k in range(NC):
        @pl.when(c == k)
        def _(p=p, k=k):
          do_pass(2 * p + k)
    if with_cnt:
      for k in range(NC):
        @pl.when(c == k)
        def _(k=k):
          do_cnt_pass(k)

  return pl.kernel(body, out_type=out_type if with_cnt else out_type[0],
                   mesh=mesh, scratch_types=scratch, interpret=interpret,
                   compiler_params=pltpu.CompilerParams(
                       use_tc_tiling_on_sc=False))


def _make_gather(BIDX, D, interpret=False):
  rows_per_w = BIDX // NW
  JW = rows_per_w // 128
  mesh = plsc.VectorSubcoreMesh(
      core_axis_name="c", subcore_axis_name="s",
      num_cores=NC, num_subcores=NS)
  scratch = [
      pltpu.VMEM((BIDX // 128, 128), jnp.int32),
      pltpu.VMEM((128, D), jnp.float32),
      pltpu.SemaphoreType.DMA,
  ]

  def body(table, idx2, out, iv, rv, sem):
    c = lax.axis_index("c")
    s = lax.axis_index("s")
    w = s * NC + c
    pltpu.sync_copy(idx2, iv)
    for j in range(JW):
      pltpu.async_copy(table.at[iv.at[w * JW + j]], rv, sem).wait()
      pltpu.sync_copy(rv, out.at[pl.ds(w * rows_per_w + j * 128, 128)])

  return pl.kernel(
      body, out_type=jax.ShapeDtypeStruct((BIDX, D), jnp.float32),
      mesh=mesh, scratch_types=scratch, interpret=interpret,
      compiler_params=pltpu.CompilerParams(use_tc_tiling_on_sc=False))


def _tc_a(S1, cnt, x, W1r, Ws1, b1, W2r, Ws2, NRP, interpret=False):
  N, D = x.shape
  H = Ws1.shape[1]
  R = W1r.shape[0]
  LAST = Ws2.shape[1]
  bn = 1000
  npb = N // bn

  def body(s0_ref, s1_ref, s2_ref, s3_ref, c0_ref, c1_ref, c2_ref, c3_ref,
           x_ref, w1_ref, ws1_ref, b1_ref, w2_ref, ws2_ref,
           g2_ref, self2_ref, h_ref):
    r = pl.program_id(1)

    @pl.when(r == 0)
    def _():
      acc = jnp.dot(x_ref[...], ws1_ref[...],
                    preferred_element_type=jnp.float32) + b1_ref[...]
      for rr, (s_ref, c_ref) in enumerate(
          zip((s0_ref, s1_ref, s2_ref, s3_ref),
              (c0_ref, c1_ref, c2_ref, c3_ref))):
        cnt_rr = c_ref[:, 0:1] + c_ref[:, 32:33]
        inv = 1.0 / jnp.maximum(cnt_rr, 1.0)
        acc = acc + jnp.dot(s_ref[...] * inv, w1_ref[rr],
                            preferred_element_type=jnp.float32)
      h = jnp.maximum(acc, 0.0)
      h_ref[...] = h
      self2_ref[...] = jnp.dot(h, ws2_ref[...],
                               preferred_element_type=jnp.float32)

    g2_ref[...] = jnp.dot(h_ref[...], w2_ref[r],
                          preferred_element_type=jnp.float32)

  sspec = [pl.BlockSpec((bn, D), lambda i, r, rr=rr: (rr * npb + i, 0))
           for rr in range(4)]
  cspec = [pl.BlockSpec((bn, 128), lambda i, r, rr=rr: (rr * npb + i, 0))
           for rr in range(4)]
  return pl.pallas_call(
      body,
      grid=(npb, R),
      in_specs=sspec + cspec + [
          pl.BlockSpec((bn, D), lambda i, r: (i, 0)),
          pl.BlockSpec((R, D, H), lambda i, r: (0, 0, 0)),
          pl.BlockSpec((D, H), lambda i, r: (0, 0)),
          pl.BlockSpec((1, H), lambda i, r: (0, 0)),
          pl.BlockSpec((R, H, LAST), lambda i, r: (0, 0, 0)),
          pl.BlockSpec((H, LAST), lambda i, r: (0, 0)),
      ],
      out_specs=[
          pl.BlockSpec((bn, LAST), lambda i, r: (r * npb + i, 0)),
          pl.BlockSpec((bn, LAST), lambda i, r: (i, 0)),
      ],
      out_shape=[
          jax.ShapeDtypeStruct((NRP, LAST), jnp.float32),
          jax.ShapeDtypeStruct((N, LAST), jnp.float32),
      ],
      scratch_shapes=[pltpu.VMEM((bn, H), jnp.float32)],
      interpret=interpret,
  )(S1, S1, S1, S1, cnt, cnt, cnt, cnt, x, W1r, Ws1, b1, W2r, Ws2)


def _tc_b(S2, cnt, self2, b2, N, interpret=False):
  LAST = self2.shape[1]
  bn = 1000
  npb = N // bn

  def body(s0_ref, s1_ref, s2_ref, s3_ref, c0_ref, c1_ref, c2_ref, c3_ref,
           self2_ref, b2_ref, out_ref):
    acc = self2_ref[...] + b2_ref[...]
    for s_ref, c_ref in zip((s0_ref, s1_ref, s2_ref, s3_ref),
                            (c0_ref, c1_ref, c2_ref, c3_ref)):
      cnt_rr = c_ref[:, 0:1] + c_ref[:, 32:33]
      inv = 1.0 / jnp.maximum(cnt_rr, 1.0)
      acc = acc + s_ref[...] * inv
    out_ref[...] = jnp.maximum(acc, 0.0)

  sspec = [pl.BlockSpec((bn, LAST), lambda i, rr=rr: (rr * npb + i, 0))
           for rr in range(4)]
  cspec = [pl.BlockSpec((bn, 128), lambda i, rr=rr: (rr * npb + i, 0))
           for rr in range(4)]
  return pl.pallas_call(
      body,
      grid=(npb,),
      in_specs=sspec + cspec + [
          pl.BlockSpec((bn, LAST), lambda i: (i, 0)),
          pl.BlockSpec((1, LAST), lambda i: (0, 0)),
      ],
      out_specs=pl.BlockSpec((bn, LAST), lambda i: (i, 0)),
      out_shape=jax.ShapeDtypeStruct((N, LAST), jnp.float32),
      interpret=interpret,
  )(S2, S2, S2, S2, cnt, cnt, cnt, cnt, self2, b2)


def _tc_c(x1, x2, Wf1a, Wf1b, bf1, Wf2, bf2, Wf3p, bf3p, interpret=False):
  B, LAST = x1.shape
  H1 = Wf1a.shape[1]
  H2 = Wf2.shape[1]
  OP = Wf3p.shape[1]
  bn = 1024
  grid = B // bn

  def body(x1_ref, x2_ref, a_ref, b_ref, b1_ref, w2_ref, b2_ref, w3_ref,
           b3_ref, out_ref):
    y = jnp.dot(x1_ref[...], a_ref[...], preferred_element_type=jnp.float32)
    y = y + jnp.dot(x2_ref[...], b_ref[...],
                    preferred_element_type=jnp.float32)
    y = jnp.maximum(y + b1_ref[...], 0.0)
    y = jnp.maximum(jnp.dot(y, w2_ref[...],
                            preferred_element_type=jnp.float32) + b2_ref[...],
                    0.0)
    out_ref[...] = jnp.dot(y, w3_ref[...],
                           preferred_element_type=jnp.float32) + b3_ref[...]

  return pl.pallas_call(
      body,
      grid=(grid,),
      in_specs=[
          pl.BlockSpec((bn, LAST), lambda i: (i, 0)),
          pl.BlockSpec((bn, LAST), lambda i: (i, 0)),
          pl.BlockSpec((LAST, H1), lambda i: (0, 0)),
          pl.BlockSpec((LAST, H1), lambda i: (0, 0)),
          pl.BlockSpec((1, H1), lambda i: (0, 0)),
          pl.BlockSpec((H1, H2), lambda i: (0, 0)),
          pl.BlockSpec((1, H2), lambda i: (0, 0)),
          pl.BlockSpec((H2, OP), lambda i: (0, 0)),
          pl.BlockSpec((1, OP), lambda i: (0, 0)),
      ],
      out_specs=pl.BlockSpec((bn, OP), lambda i: (i, 0)),
      out_shape=jax.ShapeDtypeStruct((B, OP), jnp.float32),
      interpret=interpret,
  )(x1, x2, Wf1a, Wf1b, bf1, Wf2, bf2, Wf3p, bf3p)


def kernel(inputs, edge_index, edge_type, node_feature,
           W1, Ws1, b1, W2, Ws2, b2,
           Wf1, bf1, Wf2, bf2, Wf3, bf3):
  N, D = node_feature.shape
  H = Ws1.shape[1]
  R = W1.shape[0] // D
  LAST = W2.shape[1]
  E = edge_type.shape[0]
  B = inputs.shape[0]
  NR = N * R

  NRP = -(-NR // (NS * 128)) * (NS * 128)
  EP = -(-E // (NS * 128 * 8)) * (NS * 128 * 8)
  npad = EP - E

  src = edge_index[0].astype(jnp.int32)
  dst = edge_index[1].astype(jnp.int32)
  rel = edge_type.astype(jnp.int32)

  pad_g = jnp.arange(npad, dtype=jnp.int32) % N
  pad_s = NR + jnp.arange(npad, dtype=jnp.int32) % (NRP - NR)
  relN = rel * N
  sidx = jnp.concatenate([relN + dst, pad_s]).reshape(EP // 128, 128)
  g1 = jnp.concatenate([src, pad_g]).reshape(EP // 128, 128)
  g2 = jnp.concatenate([relN + src, pad_g]).reshape(EP // 128, 128)

  z32 = jnp.zeros((NRP // NS, 32), jnp.float32)

  agg1 = _make_agg(N, EP, NRP, with_cnt=True)
  S1, cnt = agg1(node_feature.reshape(N * 4, 32), g1, sidx, z32)

  G2, self2 = _tc_a(S1, cnt, node_feature,
                    W1.reshape(R, D, H), Ws1, b1.reshape(1, H),
                    W2.reshape(R, H, LAST), Ws2, NRP)

  agg2 = _make_agg(NRP, EP, NRP, with_cnt=False)
  S2 = agg2(G2.reshape(NRP * 4, 32), g2, sidx, z32)

  h2 = _tc_b(S2, cnt, self2, b2.reshape(1, LAST), N)

  idx_all = jnp.concatenate(
      [inputs[:, 0], inputs[:, 1]]).astype(jnp.int32).reshape(-1, 128)
  rows = _make_gather(2 * B, LAST)(h2, idx_all)
  x1, x2 = rows[:B], rows[B:]

  OP = 8
  Wf3p = jnp.concatenate(
      [Wf3, jnp.zeros((Wf3.shape[0], OP - 1), jnp.float32)], axis=1)
  bf3p = jnp.concatenate(
      [bf3, jnp.zeros((OP - 1,), jnp.float32)]).reshape(1, OP)
  out = _tc_c(x1, x2, Wf1[:LAST], Wf1[LAST:], bf1.reshape(1, -1),
              Wf2, bf2.reshape(1, -1), Wf3p, bf3p)
  return out[:, 0:1]
